# baseline (speedup 1.0000x reference)
import functools

import jax
import jax.numpy as jnp
from jax import lax
from jax.experimental import pallas as pl
from jax.experimental.pallas import tpu as pltpu
from jax.experimental.pallas import tpu_sc as plsc

_B = 4096 * 200
_DC = 128
_DS = 32
_CHUNK = 128
_NW = 32


def kernel(x, x_role, content_table, struct_table):
    x_flat = x.reshape(-1).astype(jnp.int32)
    role_flat = x_role.reshape(-1).astype(jnp.int32)

    mesh = plsc.VectorSubcoreMesh(core_axis_name="c", subcore_axis_name="s")

    @functools.partial(
        pl.kernel,
        mesh=mesh,
        out_type=[
            jax.ShapeDtypeStruct((_B, _DC + _DS), jnp.float32),
            jax.ShapeDtypeStruct((_B,), jnp.int32),
        ],
        scratch_types=[
            pltpu.VMEM((_CHUNK,), jnp.int32),
            pltpu.VMEM((_CHUNK,), jnp.int32),
            pltpu.VMEM((_CHUNK, _DC), jnp.float32),
            pltpu.VMEM((_CHUNK, _DS), jnp.float32),
            pltpu.VMEM((_CHUNK,), jnp.int32),
            pltpu.SemaphoreType.DMA,
            pltpu.SemaphoreType.DMA,
        ],
        compiler_params=pltpu.CompilerParams(use_tc_tiling_on_sc=False),
    )
    def run(x_hbm, role_hbm, ct_hbm, st_hbm, out_hbm, mask_hbm,
            idx_v, role_v, content_v, struct_v, mask_v, sem_idx, sem_gat):
        wid = lax.axis_index("s") * 2 + lax.axis_index("c")
        per_w = _B // _NW
        w_base = wid * per_w

        def step(i, carry):
            base = w_base + i * _CHUNK
            cp_i = pltpu.async_copy(x_hbm.at[pl.ds(base, _CHUNK)], idx_v,
                                    sem_idx)
            cp_r = pltpu.async_copy(role_hbm.at[pl.ds(base, _CHUNK)], role_v,
                                    sem_idx)
            cp_i.wait()
            cp_r.wait()
            gc = pltpu.async_copy(ct_hbm.at[idx_v], content_v, sem_gat)
            gs = pltpu.async_copy(st_hbm.at[role_v], struct_v, sem_gat)
            for g in range(_CHUNK // 16):
                v = idx_v[pl.ds(g * 16, 16)]
                mask_v[pl.ds(g * 16, 16)] = jnp.where(
                    v != 0, jnp.int32(1), jnp.int32(0))
            pltpu.sync_copy(mask_v, mask_hbm.at[pl.ds(base, _CHUNK)])
            gc.wait()
            gs.wait()
            pltpu.sync_copy(content_v,
                            out_hbm.at[pl.ds(base, _CHUNK), pl.ds(0, _DC)])
            pltpu.sync_copy(struct_v,
                            out_hbm.at[pl.ds(base, _CHUNK), pl.ds(_DC, _DS)])
            return carry

        lax.fori_loop(0, per_w // _CHUNK, step, 0)

    out, mask_i32 = run(x_flat, role_flat, content_table, struct_table)
    ret = out.reshape(x.shape[0], x.shape[1], _DC + _DS)
    mask = mask_i32.reshape(x.shape).astype(bool)
    return (ret, mask)
